# Initial kernel scaffold; baseline (speedup 1.0000x reference)
import jax
import jax.numpy as jnp
from jax.experimental import pallas as pl


def kernel(input, weight):
    raise NotImplementedError("write your pallas kernel here")


# speedup vs baseline: 2.5359x; 2.5359x over previous
import functools

import jax
import jax.numpy as jnp
from jax import lax
from jax.experimental import pallas as pl
from jax.experimental.pallas import tpu as pltpu
from jax.experimental.pallas import tpu_sc as plsc

B = 16384
L = 50
D = 64
LANES = 16
NC, NS = 2, 16
NW = NC * NS
BAGS_PER_BLK = 2
IDX_PER_BLK = BAGS_PER_BLK * L
NBLKS_TOTAL = (B * L) // IDX_PER_BLK
BLKS_PER_W = NBLKS_TOTAL // NW
BAGS_PER_W = B // NW
NBUF = 2
NSTEPS = BLKS_PER_W // NBUF


def _make_embed_bag():
    mesh = plsc.VectorSubcoreMesh(core_axis_name="c", subcore_axis_name="s")

    @functools.partial(
        pl.kernel,
        out_type=jax.ShapeDtypeStruct((B, D), jnp.float32),
        mesh=mesh,
        compiler_params=pltpu.CompilerParams(use_tc_tiling_on_sc=False),
        scratch_types=[
            pltpu.VMEM((BLKS_PER_W, IDX_PER_BLK), jnp.int32),
            pltpu.VMEM((BAGS_PER_W, D), jnp.float32),
            pltpu.VMEM((IDX_PER_BLK, D), jnp.float32),
            pltpu.VMEM((IDX_PER_BLK, D), jnp.float32),
            pltpu.SemaphoreType.DMA,
            pltpu.SemaphoreType.DMA,
        ],
    )
    def embed_bag(idx_hbm, table_hbm, out_hbm, idx_v, out_v, rows0, rows1,
                  sem0, sem1):
        rows = (rows0, rows1)
        sems = (sem0, sem1)
        wid = lax.axis_index("s") * NC + lax.axis_index("c")

        pltpu.sync_copy(idx_hbm.at[pl.ds(wid * BLKS_PER_W, BLKS_PER_W)],
                        idx_v)

        def start(blk, j):
            pltpu.make_async_copy(table_hbm.at[idx_v.at[blk]], rows[j],
                                  sems[j]).start()

        def wait(j):
            pltpu.make_async_copy(table_hbm.at[idx_v.at[0]], rows[j],
                                  sems[j]).wait()

        def reduce_block(blk, j):
            for bag in range(BAGS_PER_BLK):
                base = bag * L
                accs = [rows[j][base, pl.ds(g * LANES, LANES)]
                        for g in range(D // LANES)]
                for r in range(1, L):
                    for g in range(D // LANES):
                        accs[g] = accs[g] + rows[j][base + r,
                                                    pl.ds(g * LANES, LANES)]
                for g in range(D // LANES):
                    out_v[blk * BAGS_PER_BLK + bag,
                          pl.ds(g * LANES, LANES)] = accs[g] * (1.0 / L)

        for j in range(NBUF):
            start(j, j)

        def body(i, carry):
            for j in range(NBUF):
                blk = i * NBUF + j
                wait(j)
                reduce_block(blk, j)
                start(blk + NBUF, j)
            return carry

        lax.fori_loop(0, NSTEPS - 1, body, 0)

        for j in range(NBUF):
            blk = (NSTEPS - 1) * NBUF + j
            wait(j)
            reduce_block(blk, j)

        pltpu.sync_copy(out_v,
                        out_hbm.at[pl.ds(wid * BAGS_PER_W, BAGS_PER_W)])

    return embed_bag


_embed_bag = _make_embed_bag()


@jax.jit
def kernel(input, weight):
    idx = input.reshape(NBLKS_TOTAL, IDX_PER_BLK)
    return _embed_bag(idx, weight)
